# Initial kernel scaffold; baseline (speedup 1.0000x reference)
import jax
import jax.numpy as jnp
from jax.experimental import pallas as pl


def kernel(indices, node_emb):
    raise NotImplementedError("write your pallas kernel here")


# speedup vs baseline: 2.1519x; 2.1519x over previous
import functools

import jax
import jax.numpy as jnp
from jax import lax
from jax.experimental import pallas as pl
from jax.experimental.pallas import tpu as pltpu
from jax.experimental.pallas import tpu_sc as plsc

EMB = 16
NCORES = 2
NSUB = 16
NW = NCORES * NSUB
CHUNK = 2048
SUBIDX = 128
KSUB = CHUNK // SUBIDX
GROUPS = CHUNK // 16


def _renorm_chunk(rows_v):
    lane = lax.iota(jnp.int32, 16)

    def g_body(g, carry):
        rowv = lane + g * 16
        cols = [
            plsc.load_gather(rows_v, [rowv, jnp.full((16,), j, jnp.int32)])
            for j in range(EMB)
        ]
        ss = cols[0] * cols[0]
        for j in range(1, EMB):
            ss = ss + cols[j] * cols[j]
        s = jnp.maximum(ss, 1.0)
        i = plsc.bitcast(s, jnp.int32)
        i = jnp.full((16,), 0x5F3759DF, jnp.int32) - lax.shift_right_logical(i, 1)
        y = plsc.bitcast(i, jnp.float32)
        for _ in range(3):
            y = y * (1.5 - 0.5 * s * y * y)
        scale = jnp.where(ss > 1.0, y, 1.0)
        for j in range(EMB):
            plsc.store_scatter(
                rows_v, [rowv, jnp.full((16,), j, jnp.int32)], cols[j] * scale
            )
        return carry

    lax.fori_loop(0, GROUPS, g_body, 0)


def kernel(indices, node_emb):
    B, H = indices.shape
    N = B * H
    per_w = N // NW
    n_chunks = per_w // CHUNK
    idx2d = indices.reshape(N // SUBIDX, SUBIDX).astype(jnp.int32)

    mesh = plsc.VectorSubcoreMesh(core_axis_name="c", subcore_axis_name="s")

    @functools.partial(
        pl.kernel,
        mesh=mesh,
        out_type=jax.ShapeDtypeStruct((N, EMB), jnp.float32),
        compiler_params=pltpu.CompilerParams(
            needs_layout_passes=False, use_tc_tiling_on_sc=False
        ),
        scratch_types=[
            pltpu.VMEM((KSUB, SUBIDX), jnp.int32),
            pltpu.VMEM((CHUNK, EMB), jnp.float32),
            pltpu.SemaphoreType.DMA,
        ],
    )
    def k(idx_hbm, table_hbm, out_hbm, idx_v, rows_v, sem):
        wid = lax.axis_index("s") * NCORES + lax.axis_index("c")
        wbase = wid * per_w

        def chunk_body(t, carry):
            cbase = wbase + t * CHUNK
            crow = pl.multiple_of(cbase // SUBIDX, 8)
            pltpu.sync_copy(idx_hbm.at[pl.ds(crow, KSUB)], idx_v)
            copies = [
                pltpu.async_copy(
                    table_hbm.at[idx_v.at[j]],
                    rows_v.at[pl.ds(j * SUBIDX, SUBIDX)],
                    sem,
                )
                for j in range(KSUB)
            ]
            for c in copies:
                c.wait()
            _renorm_chunk(rows_v)
            pltpu.sync_copy(rows_v, out_hbm.at[pl.ds(cbase, CHUNK)])
            return carry

        lax.fori_loop(0, n_chunks, chunk_body, 0)

    out = k(idx2d, node_emb)
    return out.reshape(B, H, EMB)
